# baseline (speedup 1.0000x reference)
import jax
import jax.numpy as jnp
from jax.experimental import pallas as pl
from jax.experimental.pallas import tpu as pltpu


def _pos_embed_kernel(emb_ref, out_ref, t_ref):
    b = pl.program_id(0)

    @pl.when(b == 0)
    def _():
        t_ref[...] = emb_ref[...].T

    out_ref[0] = t_ref[...]


def kernel(x, mask, row_embed):
    B = x.shape[0]
    F = x.shape[1]
    L = x.shape[-1]
    return pl.pallas_call(
        _pos_embed_kernel,
        grid=(B,),
        in_specs=[pl.BlockSpec((L, F), lambda b: (0, 0))],
        out_specs=pl.BlockSpec((1, F, L), lambda b: (b, 0, 0)),
        out_shape=jax.ShapeDtypeStruct((B, F, L), jnp.float32),
        scratch_shapes=[pltpu.VMEM((F, L), jnp.float32)],
    )(row_embed)


# speedup vs baseline: 1.1437x; 1.1437x over previous
import jax
import jax.numpy as jnp
from jax.experimental import pallas as pl
from jax.experimental.pallas import tpu as pltpu


def _pos_embed_kernel(emb_ref, out_ref, t_ref, sems):
    B = out_ref.shape[0]
    t_ref[...] = emb_ref[...].T
    copies = [
        pltpu.make_async_copy(t_ref, out_ref.at[b], sems.at[b]) for b in range(B)
    ]
    for cp in copies:
        cp.start()
    for cp in copies:
        cp.wait()


def kernel(x, mask, row_embed):
    B = x.shape[0]
    F = x.shape[1]
    L = x.shape[-1]
    return pl.pallas_call(
        _pos_embed_kernel,
        grid=(1,),
        in_specs=[pl.BlockSpec((L, F), lambda i: (0, 0))],
        out_specs=pl.BlockSpec(memory_space=pl.ANY),
        out_shape=jax.ShapeDtypeStruct((B, F, L), jnp.float32),
        scratch_shapes=[
            pltpu.VMEM((F, L), jnp.float32),
            pltpu.SemaphoreType.DMA((B,)),
        ],
    )(row_embed)
